# baseline (speedup 1.0000x reference)
import functools

import jax
import jax.numpy as jnp
from jax import lax
from jax.experimental import pallas as pl
from jax.experimental.pallas import tpu as pltpu
from jax.experimental.pallas import tpu_sc as plsc

NC = 2
NS = 16
NW = NC * NS
LANES = 16


def _largest_chunk(epw):
    for c in range(128, 7, -8):
        if epw % c == 0:
            return c
    return 8


def _round_up(v, m):
    return -(-v // m) * m


def _make_mask_deg_kernel(N, NCH, CH, NACC, STRIPE):
    KSUB = CH // LANES
    mesh = plsc.VectorSubcoreMesh(core_axis_name="c", subcore_axis_name="s",
                                  num_cores=NC, num_subcores=NS)

    @functools.partial(
        pl.kernel,
        out_type=[
            jax.ShapeDtypeStruct((NW, NCH, CH), jnp.int32),
            jax.ShapeDtypeStruct((NC, NACC, 16), jnp.float32),
        ],
        mesh=mesh,
        compiler_params=pltpu.CompilerParams(use_tc_tiling_on_sc=False),
        scratch_types=[
            pltpu.VMEM((NCH, CH), jnp.int32),
            pltpu.VMEM((NCH, CH), jnp.int32),
            pltpu.VMEM((NCH, CH), jnp.int32),
            pltpu.VMEM((NCH, CH), jnp.int32),
            pltpu.VMEM((CH, 16), jnp.float32),
            pltpu.VMEM_SHARED((NACC, 16), jnp.float32),
        ],
    )
    def mask_deg(row2d, col2d, ones8, zeros8, mcol_out, degp_out,
                 rowb, colb, mrowb, mcolb, onesb, acc):
        cid = lax.axis_index("c")
        sid = lax.axis_index("s")
        wid = sid * NC + cid
        s0 = sid * STRIPE
        SPREAD = ((NACC - N) // LANES) * LANES
        iota = lax.iota(jnp.int32, LANES)

        pltpu.sync_copy(zeros8.at[pl.ds(s0, STRIPE)], acc.at[pl.ds(s0, STRIPE)])
        pltpu.sync_copy(row2d.at[wid], rowb)
        pltpu.sync_copy(col2d.at[wid], colb)
        pltpu.sync_copy(ones8, onesb)

        def mask_body(j, _):
            for k in range(KSUB):
                sl = pl.ds(k * LANES, LANES)
                r = rowb[j, sl]
                c = colb[j, sl]
                is_self = r == c
                if SPREAD >= LANES:
                    base = lax.rem((j * KSUB + k) * LANES, SPREAD)
                    pad = (N + base) + iota
                else:
                    pad = jnp.full((LANES,), N, jnp.int32)
                mrowb[j, sl] = jnp.where(is_self, pad, r)
                mcolb[j, sl] = jnp.where(is_self, pad, c)
            return 0

        lax.fori_loop(0, NCH, mask_body, 0)
        pltpu.sync_copy(mcolb, mcol_out.at[wid])
        plsc.subcore_barrier()

        def scat_body(j, _):
            pltpu.sync_copy(onesb, acc.at[mrowb.at[j]], add=True)
            return 0

        lax.fori_loop(0, NCH, scat_body, 0)
        plsc.subcore_barrier()
        pltpu.sync_copy(acc.at[pl.ds(s0, STRIPE)],
                        degp_out.at[cid, pl.ds(s0, STRIPE)])

    return mask_deg


def _make_propagate_kernel(N, NCH, CH, D, NACC, STRIPE):
    mesh = plsc.VectorSubcoreMesh(core_axis_name="c", subcore_axis_name="s",
                                  num_cores=NC, num_subcores=NS)

    @functools.partial(
        pl.kernel,
        out_type=jax.ShapeDtypeStruct((NC, NACC, D), jnp.float32),
        mesh=mesh,
        compiler_params=pltpu.CompilerParams(use_tc_tiling_on_sc=False),
        scratch_types=[
            pltpu.VMEM((NCH, CH), jnp.int32),
            pltpu.VMEM((NCH, CH), jnp.int32),
            pltpu.VMEM((CH, D), jnp.float32),
            pltpu.VMEM((CH, D), jnp.float32),
            pltpu.SemaphoreType.DMA,
            pltpu.SemaphoreType.DMA,
            pltpu.SemaphoreType.DMA,
            pltpu.VMEM_SHARED((NACC, D), jnp.float32),
        ],
    )
    def propagate(hs_hbm, row2d, mcol2d, zerosd, part_out,
                  ridx, cidx, rows0, rows1, semg0, semg1, sems, acc):
        cid = lax.axis_index("c")
        sid = lax.axis_index("s")
        wid = sid * NC + cid
        s0 = sid * STRIPE

        pltpu.sync_copy(zerosd.at[pl.ds(s0, STRIPE)], acc.at[pl.ds(s0, STRIPE)])
        pltpu.sync_copy(row2d.at[wid], ridx)
        pltpu.sync_copy(mcol2d.at[wid], cidx)
        plsc.subcore_barrier()

        pltpu.async_copy(hs_hbm.at[ridx.at[0]], rows0, semg0)

        def step(j, _):
            even = lax.rem(j, 2) == 0

            @pl.when(even)
            def _():
                pltpu.make_async_copy(hs_hbm.at[ridx.at[j]], rows0, semg0).wait()

                @pl.when(j > 0)
                def _():
                    pltpu.make_async_copy(rows1, acc.at[cidx.at[j - 1]], sems).wait()

                @pl.when(j + 1 < NCH)
                def _():
                    pltpu.async_copy(hs_hbm.at[ridx.at[j + 1]], rows1, semg1)
                pltpu.async_copy(rows0, acc.at[cidx.at[j]], sems, add=True)

            @pl.when(jnp.logical_not(even))
            def _():
                pltpu.make_async_copy(hs_hbm.at[ridx.at[j]], rows1, semg1).wait()
                pltpu.make_async_copy(rows0, acc.at[cidx.at[j - 1]], sems).wait()

                @pl.when(j + 1 < NCH)
                def _():
                    pltpu.async_copy(hs_hbm.at[ridx.at[j + 1]], rows0, semg0)
                pltpu.async_copy(rows1, acc.at[cidx.at[j]], sems, add=True)
            return 0

        lax.fori_loop(0, NCH, step, 0)
        last_buf = rows0 if (NCH - 1) % 2 == 0 else rows1
        pltpu.make_async_copy(last_buf, acc.at[cidx.at[NCH - 1]], sems).wait()
        plsc.subcore_barrier()
        pltpu.sync_copy(acc.at[pl.ds(s0, STRIPE)],
                        part_out.at[cid, pl.ds(s0, STRIPE)])

    return propagate


def _dinv_block(degp):
    deg = degp[0, :, 0:1] + degp[1, :, 0:1] + 1.0
    return lax.rsqrt(deg)


def _tc_scale_matmul(x_ref, w_ref, degp_ref, out_ref):
    dinv = _dinv_block(degp_ref[...])
    h = jnp.dot(x_ref[...], w_ref[...], preferred_element_type=jnp.float32)
    out_ref[...] = h * dinv


def _tc_combine_matmul(part_ref, hs_ref, degp_ref, w_ref, b_ref, out_ref):
    dinv = _dinv_block(degp_ref[...])
    s = part_ref[0] + part_ref[1] + hs_ref[...]
    z = jnp.maximum(dinv * s + b_ref[...], 0.0)
    h = jnp.dot(z, w_ref[...], preferred_element_type=jnp.float32)
    out_ref[...] = h * dinv


def _tc_final(part_ref, hs_ref, degp_ref, b_ref, out_ref):
    dinv = _dinv_block(degp_ref[...])
    s = part_ref[0] + part_ref[1] + hs_ref[...]
    out_ref[...] = dinv * s + b_ref[...]


def kernel(x, edge_index, cache_name, W1, b1, W2, b2, Wd, bd):
    N, Din = x.shape
    Dh = W1.shape[1]
    Do = W2.shape[1]
    E = edge_index.shape[1]
    D = Dh

    EPW = E // NW
    CH = min(112, _round_up(EPW, 8))
    NCH = -(-EPW // CH)
    PADW = NCH * CH - EPW
    STRIPE = _round_up(-(-(N + 1) // NS), 8)
    NACC = STRIPE * NS

    row = edge_index[0].astype(jnp.int32)
    col = edge_index[1].astype(jnp.int32)
    if PADW:
        padz = jnp.zeros((NW, PADW), jnp.int32)
        row2d = jnp.concatenate([row.reshape(NW, EPW), padz], 1).reshape(NW, NCH, CH)
        col2d = jnp.concatenate([col.reshape(NW, EPW), padz], 1).reshape(NW, NCH, CH)
    else:
        row2d = row.reshape(NW, NCH, CH)
        col2d = col.reshape(NW, NCH, CH)

    ones8 = jnp.ones((CH, 16), jnp.float32)
    zeros8 = jnp.zeros((NACC, 16), jnp.float32)
    zerosd = jnp.zeros((NACC, D), jnp.float32)
    b1r = b1.reshape(1, Dh)
    b2c = (b2 + jnp.asarray(cache_name, jnp.float32)).reshape(1, Do)

    mask_deg = _make_mask_deg_kernel(N, NCH, CH, NACC, STRIPE)
    propagate = _make_propagate_kernel(N, NCH, CH, D, NACC, STRIPE)

    mcol2d, degp = mask_deg(row2d, col2d, ones8, zeros8)

    BR = 400
    grid = (N // BR,)
    degp_spec = pl.BlockSpec((NC, BR, 16), lambda i: (0, i, 0))
    row_spec = pl.BlockSpec((BR, Din), lambda i: (i, 0))
    part_spec = pl.BlockSpec((NC, BR, D), lambda i: (0, i, 0))
    w_spec = pl.BlockSpec((Din, Dh), lambda i: (0, 0))
    b_spec = pl.BlockSpec((1, Dh), lambda i: (0, 0))

    hs1 = pl.pallas_call(
        _tc_scale_matmul,
        grid=grid,
        in_specs=[row_spec, w_spec, degp_spec],
        out_specs=pl.BlockSpec((BR, Dh), lambda i: (i, 0)),
        out_shape=jax.ShapeDtypeStruct((N, Dh), jnp.float32),
    )(x, W1, degp)

    part1 = propagate(hs1, row2d, mcol2d, zerosd)

    hs2 = pl.pallas_call(
        _tc_combine_matmul,
        grid=grid,
        in_specs=[part_spec, row_spec, degp_spec, w_spec, b_spec],
        out_specs=pl.BlockSpec((BR, Do), lambda i: (i, 0)),
        out_shape=jax.ShapeDtypeStruct((N, Do), jnp.float32),
    )(part1, hs1, degp, W2, b1r)

    part2 = propagate(hs2, row2d, mcol2d, zerosd)

    out = pl.pallas_call(
        _tc_final,
        grid=grid,
        in_specs=[part_spec, row_spec, degp_spec, b_spec],
        out_specs=pl.BlockSpec((BR, Do), lambda i: (i, 0)),
        out_shape=jax.ShapeDtypeStruct((N, Do), jnp.float32),
    )(part2, hs2, degp, b2c)

    return out


# speedup vs baseline: 1.7815x; 1.7815x over previous
import functools

import jax
import jax.numpy as jnp
from jax import lax
from jax.experimental import pallas as pl
from jax.experimental.pallas import tpu as pltpu
from jax.experimental.pallas import tpu_sc as plsc

NC = 2
NS = 16
NW = NC * NS
LANES = 16


def _largest_chunk(epw):
    for c in range(128, 7, -8):
        if epw % c == 0:
            return c
    return 8


def _round_up(v, m):
    return -(-v // m) * m


def _make_mask_deg_kernel(N, NCH, CH, NACC, STRIPE):
    KSUB = CH // LANES
    mesh = plsc.VectorSubcoreMesh(core_axis_name="c", subcore_axis_name="s",
                                  num_cores=NC, num_subcores=NS)

    @functools.partial(
        pl.kernel,
        out_type=[
            jax.ShapeDtypeStruct((NW, NCH, CH), jnp.int32),
            jax.ShapeDtypeStruct((NC, NACC, 16), jnp.float32),
        ],
        mesh=mesh,
        compiler_params=pltpu.CompilerParams(use_tc_tiling_on_sc=False),
        scratch_types=[
            pltpu.VMEM((NCH, CH), jnp.int32),
            pltpu.VMEM((NCH, CH), jnp.int32),
            pltpu.VMEM((NCH, CH), jnp.int32),
            pltpu.VMEM((NCH, CH), jnp.int32),
            pltpu.VMEM((CH, 16), jnp.float32),
            pltpu.VMEM_SHARED((NACC, 16), jnp.float32),
        ],
    )
    def mask_deg(row2d, col2d, ones8, zeros8, mcol_out, degp_out,
                 rowb, colb, mrowb, mcolb, onesb, acc):
        cid = lax.axis_index("c")
        sid = lax.axis_index("s")
        wid = sid * NC + cid
        s0 = sid * STRIPE
        SPREAD = ((NACC - N) // LANES) * LANES
        iota = lax.iota(jnp.int32, LANES)

        pltpu.sync_copy(zeros8.at[pl.ds(s0, STRIPE)], acc.at[pl.ds(s0, STRIPE)])
        pltpu.sync_copy(row2d.at[wid], rowb)
        pltpu.sync_copy(col2d.at[wid], colb)
        pltpu.sync_copy(ones8, onesb)

        def mask_body(j, _):
            for k in range(KSUB):
                sl = pl.ds(k * LANES, LANES)
                r = rowb[j, sl]
                c = colb[j, sl]
                is_self = r == c
                if SPREAD >= LANES:
                    base = lax.rem((j * KSUB + k) * LANES, SPREAD)
                    pad = (N + base) + iota
                else:
                    pad = jnp.full((LANES,), N, jnp.int32)
                mrowb[j, sl] = jnp.where(is_self, pad, r)
                mcolb[j, sl] = jnp.where(is_self, pad, c)
            return 0

        lax.fori_loop(0, NCH, mask_body, 0)
        pltpu.sync_copy(mcolb, mcol_out.at[wid])
        plsc.subcore_barrier()

        def scat_body(j, _):
            pltpu.sync_copy(onesb, acc.at[mrowb.at[j]], add=True)
            return 0

        lax.fori_loop(0, NCH, scat_body, 0)
        plsc.subcore_barrier()
        pltpu.sync_copy(acc.at[pl.ds(s0, STRIPE)],
                        degp_out.at[cid, pl.ds(s0, STRIPE)])

    return mask_deg


def _make_propagate_kernel(N, NCH, CH, D, NACC, STRIPE):
    mesh = plsc.VectorSubcoreMesh(core_axis_name="c", subcore_axis_name="s",
                                  num_cores=NC, num_subcores=NS)

    @functools.partial(
        pl.kernel,
        out_type=jax.ShapeDtypeStruct((NC, NACC, D), jnp.float32),
        mesh=mesh,
        compiler_params=pltpu.CompilerParams(use_tc_tiling_on_sc=False),
        scratch_types=[
            pltpu.VMEM((NCH, CH), jnp.int32),
            pltpu.VMEM((NCH, CH), jnp.int32),
            pltpu.VMEM((CH, D), jnp.float32),
            pltpu.VMEM((CH, D), jnp.float32),
            pltpu.VMEM((CH, D), jnp.float32),
            pltpu.SemaphoreType.DMA,
            pltpu.SemaphoreType.DMA,
            pltpu.SemaphoreType.DMA,
            pltpu.SemaphoreType.DMA,
            pltpu.VMEM_SHARED((NACC, D), jnp.float32),
        ],
    )
    def propagate(hs_hbm, row2d, mcol2d, zerosd, part_out,
                  ridx, cidx, rows0, rows1, rows2, semg0, semg1, semg2, sems,
                  acc):
        cid = lax.axis_index("c")
        sid = lax.axis_index("s")
        wid = sid * NC + cid
        s0 = sid * STRIPE
        rows = (rows0, rows1, rows2)
        semg = (semg0, semg1, semg2)

        pltpu.sync_copy(zerosd.at[pl.ds(s0, STRIPE)], acc.at[pl.ds(s0, STRIPE)])
        pltpu.sync_copy(row2d.at[wid], ridx)
        pltpu.sync_copy(mcol2d.at[wid], cidx)
        plsc.subcore_barrier()

        pltpu.async_copy(hs_hbm.at[ridx.at[0]], rows0, semg0)
        if NCH > 1:
            pltpu.async_copy(hs_hbm.at[ridx.at[1]], rows1, semg1)

        def step(j, _):
            for b in range(3):

                @pl.when(lax.rem(j, 3) == b)
                def _():
                    pltpu.make_async_copy(
                        hs_hbm.at[ridx.at[j]], rows[b], semg[b]).wait()

                    @pl.when(j > 0)
                    def _():
                        pltpu.make_async_copy(
                            rows[(b + 2) % 3], acc.at[cidx.at[j - 1]],
                            sems).wait()

                    @pl.when(j + 2 < NCH)
                    def _():
                        pltpu.async_copy(
                            hs_hbm.at[ridx.at[j + 2]], rows[(b + 2) % 3],
                            semg[(b + 2) % 3])
                    pltpu.async_copy(rows[b], acc.at[cidx.at[j]], sems,
                                     add=True)
            return 0

        lax.fori_loop(0, NCH, step, 0)
        pltpu.make_async_copy(rows[(NCH - 1) % 3], acc.at[cidx.at[NCH - 1]],
                              sems).wait()
        plsc.subcore_barrier()
        pltpu.sync_copy(acc.at[pl.ds(s0, STRIPE)],
                        part_out.at[cid, pl.ds(s0, STRIPE)])

    return propagate


def _dinv_block(degp):
    deg = degp[0, :, 0:1] + degp[1, :, 0:1] + 1.0
    return lax.rsqrt(deg)


def _tc_scale_matmul(x_ref, w_ref, degp_ref, out_ref):
    dinv = _dinv_block(degp_ref[...])
    h = jnp.dot(x_ref[...], w_ref[...], preferred_element_type=jnp.float32)
    out_ref[...] = h * dinv


def _tc_combine_matmul(part_ref, hs_ref, degp_ref, w_ref, b_ref, out_ref):
    dinv = _dinv_block(degp_ref[...])
    s = part_ref[0] + part_ref[1] + hs_ref[...]
    z = jnp.maximum(dinv * s + b_ref[...], 0.0)
    h = jnp.dot(z, w_ref[...], preferred_element_type=jnp.float32)
    out_ref[...] = h * dinv


def _tc_final(part_ref, hs_ref, degp_ref, b_ref, out_ref):
    dinv = _dinv_block(degp_ref[...])
    s = part_ref[0] + part_ref[1] + hs_ref[...]
    out_ref[...] = dinv * s + b_ref[...]


def kernel(x, edge_index, cache_name, W1, b1, W2, b2, Wd, bd):
    N, Din = x.shape
    Dh = W1.shape[1]
    Do = W2.shape[1]
    E = edge_index.shape[1]
    D = Dh

    EPW = E // NW
    CH = min(80, _round_up(EPW, 8))
    NCH = -(-EPW // CH)
    PADW = NCH * CH - EPW
    STRIPE = -(-(N + 1) // NS)
    NACC = STRIPE * NS

    row = edge_index[0].astype(jnp.int32)
    col = edge_index[1].astype(jnp.int32)
    if PADW:
        padz = jnp.zeros((NW, PADW), jnp.int32)
        row2d = jnp.concatenate([row.reshape(NW, EPW), padz], 1).reshape(NW, NCH, CH)
        col2d = jnp.concatenate([col.reshape(NW, EPW), padz], 1).reshape(NW, NCH, CH)
    else:
        row2d = row.reshape(NW, NCH, CH)
        col2d = col.reshape(NW, NCH, CH)

    ones8 = jnp.ones((CH, 16), jnp.float32)
    zeros8 = jnp.zeros((NACC, 16), jnp.float32)
    zerosd = jnp.zeros((NACC, D), jnp.float32)
    b1r = b1.reshape(1, Dh)
    b2c = (b2 + jnp.asarray(cache_name, jnp.float32)).reshape(1, Do)

    mask_deg = _make_mask_deg_kernel(N, NCH, CH, NACC, STRIPE)
    propagate = _make_propagate_kernel(N, NCH, CH, D, NACC, STRIPE)

    mcol2d, degp = mask_deg(row2d, col2d, ones8, zeros8)

    BR = 400
    grid = (N // BR,)
    degp_spec = pl.BlockSpec((NC, BR, 16), lambda i: (0, i, 0))
    row_spec = pl.BlockSpec((BR, Din), lambda i: (i, 0))
    part_spec = pl.BlockSpec((NC, BR, D), lambda i: (0, i, 0))
    w_spec = pl.BlockSpec((Din, Dh), lambda i: (0, 0))
    b_spec = pl.BlockSpec((1, Dh), lambda i: (0, 0))

    hs1 = pl.pallas_call(
        _tc_scale_matmul,
        grid=grid,
        in_specs=[row_spec, w_spec, degp_spec],
        out_specs=pl.BlockSpec((BR, Dh), lambda i: (i, 0)),
        out_shape=jax.ShapeDtypeStruct((N, Dh), jnp.float32),
    )(x, W1, degp)

    part1 = propagate(hs1, row2d, mcol2d, zerosd)

    hs2 = pl.pallas_call(
        _tc_combine_matmul,
        grid=grid,
        in_specs=[part_spec, row_spec, degp_spec, w_spec, b_spec],
        out_specs=pl.BlockSpec((BR, Do), lambda i: (i, 0)),
        out_shape=jax.ShapeDtypeStruct((N, Do), jnp.float32),
    )(part1, hs1, degp, W2, b1r)

    part2 = propagate(hs2, row2d, mcol2d, zerosd)

    out = pl.pallas_call(
        _tc_final,
        grid=grid,
        in_specs=[part_spec, row_spec, degp_spec, b_spec],
        out_specs=pl.BlockSpec((BR, Do), lambda i: (i, 0)),
        out_shape=jax.ShapeDtypeStruct((N, Do), jnp.float32),
    )(part2, hs2, degp, b2c)

    return out


# speedup vs baseline: 1.8094x; 1.0157x over previous
import functools

import jax
import jax.numpy as jnp
from jax import lax
from jax.experimental import pallas as pl
from jax.experimental.pallas import tpu as pltpu
from jax.experimental.pallas import tpu_sc as plsc

NC = 2
NS = 16
NW = NC * NS
LANES = 16


def _largest_chunk(epw):
    for c in range(128, 7, -8):
        if epw % c == 0:
            return c
    return 8


def _round_up(v, m):
    return -(-v // m) * m


def _make_mask_deg_kernel(N, NCH, CH, NACC, STRIPE):
    KSUB = CH // LANES
    mesh = plsc.VectorSubcoreMesh(core_axis_name="c", subcore_axis_name="s",
                                  num_cores=NC, num_subcores=NS)

    @functools.partial(
        pl.kernel,
        out_type=[
            jax.ShapeDtypeStruct((NW, NCH, CH), jnp.int32),
            jax.ShapeDtypeStruct((NC, NACC, 16), jnp.float32),
        ],
        mesh=mesh,
        compiler_params=pltpu.CompilerParams(use_tc_tiling_on_sc=False),
        scratch_types=[
            pltpu.VMEM((NCH, CH), jnp.int32),
            pltpu.VMEM((NCH, CH), jnp.int32),
            pltpu.VMEM((NCH, CH), jnp.int32),
            pltpu.VMEM((NCH, CH), jnp.int32),
            pltpu.VMEM((CH, 16), jnp.float32),
            pltpu.SemaphoreType.DMA,
            pltpu.VMEM_SHARED((NACC, 16), jnp.float32),
        ],
    )
    def mask_deg(row2d, col2d, ones8, zeros8, mcol_out, degp_out,
                 rowb, colb, mrowb, mcolb, onesb, semz, acc):
        cid = lax.axis_index("c")
        sid = lax.axis_index("s")
        wid = sid * NC + cid
        s0 = sid * STRIPE
        SPREAD = ((NACC - N) // LANES) * LANES
        iota = lax.iota(jnp.int32, LANES)

        z = pltpu.async_copy(zeros8.at[pl.ds(s0, STRIPE)],
                             acc.at[pl.ds(s0, STRIPE)], semz)
        r = pltpu.async_copy(row2d.at[wid], rowb, semz)
        c = pltpu.async_copy(col2d.at[wid], colb, semz)
        o = pltpu.async_copy(ones8, onesb, semz)
        z.wait()
        r.wait()
        c.wait()
        o.wait()

        def mask_body(j, _):
            for k in range(KSUB):
                sl = pl.ds(k * LANES, LANES)
                r = rowb[j, sl]
                c = colb[j, sl]
                is_self = r == c
                if SPREAD >= LANES:
                    base = lax.rem((j * KSUB + k) * LANES, SPREAD)
                    pad = (N + base) + iota
                else:
                    pad = jnp.full((LANES,), N, jnp.int32)
                mrowb[j, sl] = jnp.where(is_self, pad, r)
                mcolb[j, sl] = jnp.where(is_self, pad, c)
            return 0

        lax.fori_loop(0, NCH, mask_body, 0)
        pltpu.sync_copy(mcolb, mcol_out.at[wid])
        plsc.subcore_barrier()

        def scat_body(j, _):
            pltpu.sync_copy(onesb, acc.at[mrowb.at[j]], add=True)
            return 0

        lax.fori_loop(0, NCH, scat_body, 0)
        plsc.subcore_barrier()
        pltpu.sync_copy(acc.at[pl.ds(s0, STRIPE)],
                        degp_out.at[cid, pl.ds(s0, STRIPE)])

    return mask_deg


def _make_propagate_kernel(N, NCH, CH, D, NACC, STRIPE):
    mesh = plsc.VectorSubcoreMesh(core_axis_name="c", subcore_axis_name="s",
                                  num_cores=NC, num_subcores=NS)

    @functools.partial(
        pl.kernel,
        out_type=jax.ShapeDtypeStruct((NC, NACC, D), jnp.float32),
        mesh=mesh,
        compiler_params=pltpu.CompilerParams(use_tc_tiling_on_sc=False),
        scratch_types=[
            pltpu.VMEM((NCH, CH), jnp.int32),
            pltpu.VMEM((NCH, CH), jnp.int32),
            pltpu.VMEM((CH, D), jnp.float32),
            pltpu.VMEM((CH, D), jnp.float32),
            pltpu.VMEM((CH, D), jnp.float32),
            pltpu.SemaphoreType.DMA,
            pltpu.SemaphoreType.DMA,
            pltpu.SemaphoreType.DMA,
            pltpu.SemaphoreType.DMA,
            pltpu.VMEM_SHARED((NACC, D), jnp.float32),
        ],
    )
    def propagate(hs_hbm, row2d, mcol2d, zerosd, part_out,
                  ridx, cidx, rows0, rows1, rows2, semg0, semg1, semg2, sems,
                  acc):
        cid = lax.axis_index("c")
        sid = lax.axis_index("s")
        wid = sid * NC + cid
        s0 = sid * STRIPE
        rows = (rows0, rows1, rows2)
        semg = (semg0, semg1, semg2)

        z = pltpu.async_copy(zerosd.at[pl.ds(s0, STRIPE)],
                             acc.at[pl.ds(s0, STRIPE)], sems)
        r = pltpu.async_copy(row2d.at[wid], ridx, semg0)
        c = pltpu.async_copy(mcol2d.at[wid], cidx, semg1)
        z.wait()
        r.wait()
        c.wait()
        plsc.subcore_barrier()

        pltpu.async_copy(hs_hbm.at[ridx.at[0]], rows0, semg0)
        if NCH > 1:
            pltpu.async_copy(hs_hbm.at[ridx.at[1]], rows1, semg1)

        def step(j, _):
            for b in range(3):

                @pl.when(lax.rem(j, 3) == b)
                def _():
                    pltpu.make_async_copy(
                        hs_hbm.at[ridx.at[j]], rows[b], semg[b]).wait()

                    @pl.when(j > 0)
                    def _():
                        pltpu.make_async_copy(
                            rows[(b + 2) % 3], acc.at[cidx.at[j - 1]],
                            sems).wait()

                    @pl.when(j + 2 < NCH)
                    def _():
                        pltpu.async_copy(
                            hs_hbm.at[ridx.at[j + 2]], rows[(b + 2) % 3],
                            semg[(b + 2) % 3])
                    pltpu.async_copy(rows[b], acc.at[cidx.at[j]], sems,
                                     add=True)
            return 0

        lax.fori_loop(0, NCH, step, 0)
        pltpu.make_async_copy(rows[(NCH - 1) % 3], acc.at[cidx.at[NCH - 1]],
                              sems).wait()
        plsc.subcore_barrier()
        pltpu.sync_copy(acc.at[pl.ds(s0, STRIPE)],
                        part_out.at[cid, pl.ds(s0, STRIPE)])

    return propagate


def _dinv_block(degp):
    deg = degp[0, :, 0:1] + degp[1, :, 0:1] + 1.0
    return lax.rsqrt(deg)


def _tc_scale_matmul(x_ref, w_ref, degp_ref, out_ref):
    dinv = _dinv_block(degp_ref[...])
    h = jnp.dot(x_ref[...], w_ref[...], preferred_element_type=jnp.float32)
    out_ref[...] = h * dinv


def _tc_combine_matmul(part_ref, hs_ref, degp_ref, w_ref, b_ref, out_ref):
    dinv = _dinv_block(degp_ref[...])
    s = part_ref[0] + part_ref[1] + hs_ref[...]
    z = jnp.maximum(dinv * s + b_ref[...], 0.0)
    h = jnp.dot(z, w_ref[...], preferred_element_type=jnp.float32)
    out_ref[...] = h * dinv


def _tc_final(part_ref, hs_ref, degp_ref, b_ref, out_ref):
    dinv = _dinv_block(degp_ref[...])
    s = part_ref[0] + part_ref[1] + hs_ref[...]
    out_ref[...] = dinv * s + b_ref[...]


def kernel(x, edge_index, cache_name, W1, b1, W2, b2, Wd, bd):
    N, Din = x.shape
    Dh = W1.shape[1]
    Do = W2.shape[1]
    E = edge_index.shape[1]
    D = Dh

    EPW = E // NW
    CH = min(80, _round_up(EPW, 8))
    NCH = -(-EPW // CH)
    PADW = NCH * CH - EPW
    STRIPE = -(-(N + 1) // NS)
    NACC = STRIPE * NS

    row = edge_index[0].astype(jnp.int32)
    col = edge_index[1].astype(jnp.int32)
    if PADW:
        padz = jnp.zeros((NW, PADW), jnp.int32)
        row2d = jnp.concatenate([row.reshape(NW, EPW), padz], 1).reshape(NW, NCH, CH)
        col2d = jnp.concatenate([col.reshape(NW, EPW), padz], 1).reshape(NW, NCH, CH)
    else:
        row2d = row.reshape(NW, NCH, CH)
        col2d = col.reshape(NW, NCH, CH)

    ones8 = jnp.ones((CH, 16), jnp.float32)
    zeros8 = jnp.zeros((NACC, 16), jnp.float32)
    zerosd = jnp.zeros((NACC, D), jnp.float32)
    b1r = b1.reshape(1, Dh)
    b2c = (b2 + jnp.asarray(cache_name, jnp.float32)).reshape(1, Do)

    mask_deg = _make_mask_deg_kernel(N, NCH, CH, NACC, STRIPE)
    propagate = _make_propagate_kernel(N, NCH, CH, D, NACC, STRIPE)

    mcol2d, degp = mask_deg(row2d, col2d, ones8, zeros8)

    BR = 400
    grid = (N // BR,)
    degp_spec = pl.BlockSpec((NC, BR, 16), lambda i: (0, i, 0))
    row_spec = pl.BlockSpec((BR, Din), lambda i: (i, 0))
    part_spec = pl.BlockSpec((NC, BR, D), lambda i: (0, i, 0))
    w_spec = pl.BlockSpec((Din, Dh), lambda i: (0, 0))
    b_spec = pl.BlockSpec((1, Dh), lambda i: (0, 0))

    hs1 = pl.pallas_call(
        _tc_scale_matmul,
        grid=grid,
        in_specs=[row_spec, w_spec, degp_spec],
        out_specs=pl.BlockSpec((BR, Dh), lambda i: (i, 0)),
        out_shape=jax.ShapeDtypeStruct((N, Dh), jnp.float32),
    )(x, W1, degp)

    part1 = propagate(hs1, row2d, mcol2d, zerosd)

    hs2 = pl.pallas_call(
        _tc_combine_matmul,
        grid=grid,
        in_specs=[part_spec, row_spec, degp_spec, w_spec, b_spec],
        out_specs=pl.BlockSpec((BR, Do), lambda i: (i, 0)),
        out_shape=jax.ShapeDtypeStruct((N, Do), jnp.float32),
    )(part1, hs1, degp, W2, b1r)

    part2 = propagate(hs2, row2d, mcol2d, zerosd)

    out = pl.pallas_call(
        _tc_final,
        grid=grid,
        in_specs=[part_spec, row_spec, degp_spec, b_spec],
        out_specs=pl.BlockSpec((BR, Do), lambda i: (i, 0)),
        out_shape=jax.ShapeDtypeStruct((N, Do), jnp.float32),
    )(part2, hs2, degp, b2c)

    return out


# speedup vs baseline: 1.8101x; 1.0004x over previous
import functools

import jax
import jax.numpy as jnp
from jax import lax
from jax.experimental import pallas as pl
from jax.experimental.pallas import tpu as pltpu
from jax.experimental.pallas import tpu_sc as plsc

NC = 2
NS = 16
NW = NC * NS
LANES = 16


def _largest_chunk(epw):
    for c in range(128, 7, -8):
        if epw % c == 0:
            return c
    return 8


def _round_up(v, m):
    return -(-v // m) * m


def _make_mask_deg_kernel(N, NCH, CH, NACC, STRIPE):
    KSUB = CH // LANES
    mesh = plsc.VectorSubcoreMesh(core_axis_name="c", subcore_axis_name="s",
                                  num_cores=NC, num_subcores=NS)

    @functools.partial(
        pl.kernel,
        out_type=[
            jax.ShapeDtypeStruct((NW, NCH, CH), jnp.int32),
            jax.ShapeDtypeStruct((NC, NACC, 16), jnp.float32),
        ],
        mesh=mesh,
        compiler_params=pltpu.CompilerParams(use_tc_tiling_on_sc=False),
        scratch_types=[
            pltpu.VMEM((NCH, CH), jnp.int32),
            pltpu.VMEM((NCH, CH), jnp.int32),
            pltpu.VMEM((NCH, CH), jnp.int32),
            pltpu.VMEM((NCH, CH), jnp.int32),
            pltpu.VMEM((CH, 16), jnp.float32),
            pltpu.SemaphoreType.DMA,
            pltpu.VMEM_SHARED((NACC, 16), jnp.float32),
        ],
    )
    def mask_deg(row2d, col2d, ones8, zeros8, mcol_out, degp_out,
                 rowb, colb, mrowb, mcolb, onesb, semz, acc):
        cid = lax.axis_index("c")
        sid = lax.axis_index("s")
        wid = sid * NC + cid
        s0 = sid * STRIPE
        SPREAD = ((NACC - N) // LANES) * LANES
        iota = lax.iota(jnp.int32, LANES)

        z = pltpu.async_copy(zeros8.at[pl.ds(s0, STRIPE)],
                             acc.at[pl.ds(s0, STRIPE)], semz)
        r = pltpu.async_copy(row2d.at[wid], rowb, semz)
        c = pltpu.async_copy(col2d.at[wid], colb, semz)
        o = pltpu.async_copy(ones8, onesb, semz)
        z.wait()
        r.wait()
        c.wait()
        o.wait()

        def mask_body(j, _):
            for k in range(KSUB):
                sl = pl.ds(k * LANES, LANES)
                r = rowb[j, sl]
                c = colb[j, sl]
                is_self = r == c
                if SPREAD >= LANES:
                    base = lax.rem((j * KSUB + k) * LANES, SPREAD)
                    pad = (N + base) + iota
                else:
                    pad = jnp.full((LANES,), N, jnp.int32)
                mrowb[j, sl] = jnp.where(is_self, pad, r)
                mcolb[j, sl] = jnp.where(is_self, pad, c)
            return 0

        lax.fori_loop(0, NCH, mask_body, 0)
        pltpu.sync_copy(mcolb, mcol_out.at[wid])
        plsc.subcore_barrier()

        def scat_body(j, _):
            pltpu.sync_copy(onesb, acc.at[mrowb.at[j]], add=True)
            return 0

        lax.fori_loop(0, NCH, scat_body, 0)
        plsc.subcore_barrier()
        pltpu.sync_copy(acc.at[pl.ds(s0, STRIPE)],
                        degp_out.at[cid, pl.ds(s0, STRIPE)])

    return mask_deg


def _make_propagate_kernel(N, NCH, CH, D, NACC, STRIPE):
    mesh = plsc.VectorSubcoreMesh(core_axis_name="c", subcore_axis_name="s",
                                  num_cores=NC, num_subcores=NS)

    @functools.partial(
        pl.kernel,
        out_type=jax.ShapeDtypeStruct((NC, NACC, D), jnp.float32),
        mesh=mesh,
        compiler_params=pltpu.CompilerParams(use_tc_tiling_on_sc=False),
        scratch_types=[
            pltpu.VMEM((NCH, CH), jnp.int32),
            pltpu.VMEM((NCH, CH), jnp.int32),
            pltpu.VMEM((CH, D), jnp.float32),
            pltpu.VMEM((CH, D), jnp.float32),
            pltpu.VMEM((CH, D), jnp.float32),
            pltpu.SemaphoreType.DMA,
            pltpu.SemaphoreType.DMA,
            pltpu.SemaphoreType.DMA,
            pltpu.SemaphoreType.DMA,
            pltpu.VMEM_SHARED((NACC, D), jnp.float32),
        ],
    )
    def propagate(hs_hbm, row2d, mcol2d, zerosd, part_out,
                  ridx, cidx, rows0, rows1, rows2, semg0, semg1, semg2, sems,
                  acc):
        cid = lax.axis_index("c")
        sid = lax.axis_index("s")
        wid = sid * NC + cid
        s0 = sid * STRIPE
        rows = (rows0, rows1, rows2)
        semg = (semg0, semg1, semg2)

        z = pltpu.async_copy(zerosd.at[pl.ds(s0, STRIPE)],
                             acc.at[pl.ds(s0, STRIPE)], sems)
        r = pltpu.async_copy(row2d.at[wid], ridx, semg0)
        c = pltpu.async_copy(mcol2d.at[wid], cidx, semg1)
        z.wait()
        r.wait()
        c.wait()
        plsc.subcore_barrier()

        pltpu.async_copy(hs_hbm.at[ridx.at[0]], rows0, semg0)
        if NCH > 1:
            pltpu.async_copy(hs_hbm.at[ridx.at[1]], rows1, semg1)

        def step(j, _):
            for b in range(3):

                @pl.when(lax.rem(j, 3) == b)
                def _():
                    pltpu.make_async_copy(
                        hs_hbm.at[ridx.at[j]], rows[b], semg[b]).wait()

                    @pl.when(j > 0)
                    def _():
                        pltpu.make_async_copy(
                            rows[(b + 2) % 3], acc.at[cidx.at[j - 1]],
                            sems).wait()

                    @pl.when(j + 2 < NCH)
                    def _():
                        pltpu.async_copy(
                            hs_hbm.at[ridx.at[j + 2]], rows[(b + 2) % 3],
                            semg[(b + 2) % 3])
                    half = CH // 2
                    pltpu.async_copy(rows[b].at[pl.ds(0, half)],
                                     acc.at[cidx.at[j, pl.ds(0, half)]],
                                     sems, add=True)
                    pltpu.async_copy(rows[b].at[pl.ds(half, half)],
                                     acc.at[cidx.at[j, pl.ds(half, half)]],
                                     sems, add=True)
            return 0

        lax.fori_loop(0, NCH, step, 0)
        pltpu.make_async_copy(rows[(NCH - 1) % 3], acc.at[cidx.at[NCH - 1]],
                              sems).wait()
        plsc.subcore_barrier()
        pltpu.sync_copy(acc.at[pl.ds(s0, STRIPE)],
                        part_out.at[cid, pl.ds(s0, STRIPE)])

    return propagate


def _dinv_block(degp):
    deg = degp[0, :, 0:1] + degp[1, :, 0:1] + 1.0
    return lax.rsqrt(deg)


def _tc_scale_matmul(x_ref, w_ref, degp_ref, out_ref):
    dinv = _dinv_block(degp_ref[...])
    h = jnp.dot(x_ref[...], w_ref[...], preferred_element_type=jnp.float32)
    out_ref[...] = h * dinv


def _tc_combine_matmul(part_ref, hs_ref, degp_ref, w_ref, b_ref, out_ref):
    dinv = _dinv_block(degp_ref[...])
    s = part_ref[0] + part_ref[1] + hs_ref[...]
    z = jnp.maximum(dinv * s + b_ref[...], 0.0)
    h = jnp.dot(z, w_ref[...], preferred_element_type=jnp.float32)
    out_ref[...] = h * dinv


def _tc_final(part_ref, hs_ref, degp_ref, b_ref, out_ref):
    dinv = _dinv_block(degp_ref[...])
    s = part_ref[0] + part_ref[1] + hs_ref[...]
    out_ref[...] = dinv * s + b_ref[...]


def kernel(x, edge_index, cache_name, W1, b1, W2, b2, Wd, bd):
    N, Din = x.shape
    Dh = W1.shape[1]
    Do = W2.shape[1]
    E = edge_index.shape[1]
    D = Dh

    EPW = E // NW
    CH = min(80, _round_up(EPW, 8))
    NCH = -(-EPW // CH)
    PADW = NCH * CH - EPW
    STRIPE = -(-(N + 1) // NS)
    NACC = STRIPE * NS

    row = edge_index[0].astype(jnp.int32)
    col = edge_index[1].astype(jnp.int32)
    if PADW:
        padz = jnp.zeros((NW, PADW), jnp.int32)
        row2d = jnp.concatenate([row.reshape(NW, EPW), padz], 1).reshape(NW, NCH, CH)
        col2d = jnp.concatenate([col.reshape(NW, EPW), padz], 1).reshape(NW, NCH, CH)
    else:
        row2d = row.reshape(NW, NCH, CH)
        col2d = col.reshape(NW, NCH, CH)

    ones8 = jnp.ones((CH, 16), jnp.float32)
    zeros8 = jnp.zeros((NACC, 16), jnp.float32)
    zerosd = jnp.zeros((NACC, D), jnp.float32)
    b1r = b1.reshape(1, Dh)
    b2c = (b2 + jnp.asarray(cache_name, jnp.float32)).reshape(1, Do)

    mask_deg = _make_mask_deg_kernel(N, NCH, CH, NACC, STRIPE)
    propagate = _make_propagate_kernel(N, NCH, CH, D, NACC, STRIPE)

    mcol2d, degp = mask_deg(row2d, col2d, ones8, zeros8)

    BR = 400
    grid = (N // BR,)
    degp_spec = pl.BlockSpec((NC, BR, 16), lambda i: (0, i, 0))
    row_spec = pl.BlockSpec((BR, Din), lambda i: (i, 0))
    part_spec = pl.BlockSpec((NC, BR, D), lambda i: (0, i, 0))
    w_spec = pl.BlockSpec((Din, Dh), lambda i: (0, 0))
    b_spec = pl.BlockSpec((1, Dh), lambda i: (0, 0))

    hs1 = pl.pallas_call(
        _tc_scale_matmul,
        grid=grid,
        in_specs=[row_spec, w_spec, degp_spec],
        out_specs=pl.BlockSpec((BR, Dh), lambda i: (i, 0)),
        out_shape=jax.ShapeDtypeStruct((N, Dh), jnp.float32),
    )(x, W1, degp)

    part1 = propagate(hs1, row2d, mcol2d, zerosd)

    hs2 = pl.pallas_call(
        _tc_combine_matmul,
        grid=grid,
        in_specs=[part_spec, row_spec, degp_spec, w_spec, b_spec],
        out_specs=pl.BlockSpec((BR, Do), lambda i: (i, 0)),
        out_shape=jax.ShapeDtypeStruct((N, Do), jnp.float32),
    )(part1, hs1, degp, W2, b1r)

    part2 = propagate(hs2, row2d, mcol2d, zerosd)

    out = pl.pallas_call(
        _tc_final,
        grid=grid,
        in_specs=[part_spec, row_spec, degp_spec, b_spec],
        out_specs=pl.BlockSpec((BR, Do), lambda i: (i, 0)),
        out_shape=jax.ShapeDtypeStruct((N, Do), jnp.float32),
    )(part2, hs2, degp, b2c)

    return out


# speedup vs baseline: 1.9544x; 1.0797x over previous
import functools

import jax
import jax.numpy as jnp
from jax import lax
from jax.experimental import pallas as pl
from jax.experimental.pallas import tpu as pltpu
from jax.experimental.pallas import tpu_sc as plsc

NC = 2
NS = 16
NW = NC * NS
LANES = 16


def _largest_chunk(epw):
    for c in range(128, 7, -8):
        if epw % c == 0:
            return c
    return 8


def _round_up(v, m):
    return -(-v // m) * m


def _make_mask_deg_kernel(N, NCH, CH, NACC, STRIPE):
    KSUB = CH // LANES
    mesh = plsc.VectorSubcoreMesh(core_axis_name="c", subcore_axis_name="s",
                                  num_cores=NC, num_subcores=NS)

    @functools.partial(
        pl.kernel,
        out_type=[
            jax.ShapeDtypeStruct((NW, NCH, CH), jnp.int32),
            jax.ShapeDtypeStruct((NC, NACC, 16), jnp.float32),
        ],
        mesh=mesh,
        compiler_params=pltpu.CompilerParams(use_tc_tiling_on_sc=False),
        scratch_types=[
            pltpu.VMEM((NCH, CH), jnp.int32),
            pltpu.VMEM((NCH, CH), jnp.int32),
            pltpu.VMEM((NCH, CH), jnp.int32),
            pltpu.VMEM((NCH, CH), jnp.int32),
            pltpu.VMEM((CH, 16), jnp.float32),
            pltpu.SemaphoreType.DMA,
            pltpu.VMEM_SHARED((NACC, 16), jnp.float32),
        ],
    )
    def mask_deg(row2d, col2d, ones8, zeros8, mcol_out, degp_out,
                 rowb, colb, mrowb, mcolb, onesb, semz, acc):
        cid = lax.axis_index("c")
        sid = lax.axis_index("s")
        wid = sid * NC + cid
        s0 = sid * STRIPE
        SPREAD = ((NACC - N) // LANES) * LANES
        iota = lax.iota(jnp.int32, LANES)

        z = pltpu.async_copy(zeros8.at[pl.ds(s0, STRIPE)],
                             acc.at[pl.ds(s0, STRIPE)], semz)
        r = pltpu.async_copy(row2d.at[wid], rowb, semz)
        c = pltpu.async_copy(col2d.at[wid], colb, semz)
        o = pltpu.async_copy(ones8, onesb, semz)
        z.wait()
        r.wait()
        c.wait()
        o.wait()

        def mask_body(j, _):
            for k in range(KSUB):
                sl = pl.ds(k * LANES, LANES)
                r = rowb[j, sl]
                c = colb[j, sl]
                is_self = r == c
                if SPREAD >= LANES:
                    base = lax.rem((j * KSUB + k) * LANES, SPREAD)
                    pad = (N + base) + iota
                else:
                    pad = jnp.full((LANES,), N, jnp.int32)
                mrowb[j, sl] = jnp.where(is_self, pad, r)
                mcolb[j, sl] = jnp.where(is_self, pad, c)
            return 0

        lax.fori_loop(0, NCH, mask_body, 0)
        pltpu.sync_copy(mcolb, mcol_out.at[wid])
        plsc.subcore_barrier()

        def scat_body(j, _):
            pltpu.sync_copy(onesb, acc.at[mrowb.at[j]], add=True)
            return 0

        lax.fori_loop(0, NCH, scat_body, 0)
        plsc.subcore_barrier()
        pltpu.sync_copy(acc.at[pl.ds(s0, STRIPE)],
                        degp_out.at[cid, pl.ds(s0, STRIPE)])

    return mask_deg


def _make_propagate_kernel(N, NCH, CH, D, NACC, STRIPE):
    mesh = plsc.VectorSubcoreMesh(core_axis_name="c", subcore_axis_name="s",
                                  num_cores=NC, num_subcores=NS)

    @functools.partial(
        pl.kernel,
        out_type=jax.ShapeDtypeStruct((NC, NACC, D), jnp.float32),
        mesh=mesh,
        compiler_params=pltpu.CompilerParams(use_tc_tiling_on_sc=False),
        scratch_types=[
            pltpu.VMEM((NCH, CH), jnp.int32),
            pltpu.VMEM((NCH, CH), jnp.int32),
            pltpu.VMEM((CH, D), jnp.float32),
            pltpu.VMEM((CH, D), jnp.float32),
            pltpu.VMEM((CH, D), jnp.float32),
            pltpu.SemaphoreType.DMA,
            pltpu.SemaphoreType.DMA,
            pltpu.SemaphoreType.DMA,
            pltpu.SemaphoreType.DMA,
            pltpu.VMEM_SHARED((NACC, D), jnp.float32),
        ],
    )
    def propagate(hs_hbm, row2d, mcol2d, zerosd, part_out,
                  ridx, cidx, rows0, rows1, rows2, semg0, semg1, semg2, sems,
                  acc):
        cid = lax.axis_index("c")
        sid = lax.axis_index("s")
        wid = sid * NC + cid
        s0 = sid * STRIPE
        rows = (rows0, rows1, rows2)
        semg = (semg0, semg1, semg2)

        z = pltpu.async_copy(zerosd.at[pl.ds(s0, STRIPE)],
                             acc.at[pl.ds(s0, STRIPE)], sems)
        r = pltpu.async_copy(row2d.at[wid], ridx, semg0)
        c = pltpu.async_copy(mcol2d.at[wid], cidx, semg1)
        z.wait()
        r.wait()
        c.wait()
        plsc.subcore_barrier()

        pltpu.async_copy(hs_hbm.at[ridx.at[0]], rows0, semg0)
        if NCH > 1:
            pltpu.async_copy(hs_hbm.at[ridx.at[1]], rows1, semg1)

        def step(j, _):
            for b in range(3):

                @pl.when(lax.rem(j, 3) == b)
                def _():
                    pltpu.make_async_copy(
                        hs_hbm.at[ridx.at[j]], rows[b], semg[b]).wait()

                    @pl.when(j > 0)
                    def _():
                        pltpu.make_async_copy(
                            rows[(b + 2) % 3], acc.at[cidx.at[j - 1]],
                            sems).wait()

                    @pl.when(j + 2 < NCH)
                    def _():
                        pltpu.async_copy(
                            hs_hbm.at[ridx.at[j + 2]], rows[(b + 2) % 3],
                            semg[(b + 2) % 3])
                    pltpu.async_copy(rows[b], acc.at[cidx.at[j]], sems,
                                     add=True)
            return 0

        lax.fori_loop(0, NCH, step, 0)
        pltpu.make_async_copy(rows[(NCH - 1) % 3], acc.at[cidx.at[NCH - 1]],
                              sems).wait()
        plsc.subcore_barrier()
        pltpu.sync_copy(acc.at[pl.ds(s0, STRIPE)],
                        part_out.at[cid, pl.ds(s0, STRIPE)])

    return propagate


def _dinv_block(degp):
    deg = degp[0, :, 0:1] + degp[1, :, 0:1] + 1.0
    return lax.rsqrt(deg)


def _tc_scale_matmul(x_ref, w_ref, degp_ref, out_ref):
    dinv = _dinv_block(degp_ref[...])
    h = jnp.dot(x_ref[...], w_ref[...], preferred_element_type=jnp.float32)
    out_ref[...] = h * dinv


def _tc_combine_matmul(part_ref, hs_ref, degp_ref, w_ref, b_ref, out_ref):
    dinv = _dinv_block(degp_ref[...])
    s = part_ref[0] + part_ref[1] + hs_ref[...]
    z = jnp.maximum(dinv * s + b_ref[...], 0.0)
    h = jnp.dot(z, w_ref[...], preferred_element_type=jnp.float32)
    out_ref[...] = h * dinv


def _tc_final(part_ref, hs_ref, degp_ref, b_ref, out_ref):
    dinv = _dinv_block(degp_ref[...])
    s = part_ref[0] + part_ref[1] + hs_ref[...]
    out_ref[...] = dinv * s + b_ref[...]


def kernel(x, edge_index, cache_name, W1, b1, W2, b2, Wd, bd):
    N, Din = x.shape
    Dh = W1.shape[1]
    Do = W2.shape[1]
    E = edge_index.shape[1]
    D = Dh

    EPW = E // NW
    CH = min(80, _round_up(EPW, 8))
    NCH = -(-EPW // CH)
    PADW = NCH * CH - EPW
    STRIPE = -(-(N + 1) // NS)
    NACC = STRIPE * NS

    row = edge_index[0].astype(jnp.int32)
    col = edge_index[1].astype(jnp.int32)
    if PADW:
        padz = jnp.zeros((NW, PADW), jnp.int32)
        row2d = jnp.concatenate([row.reshape(NW, EPW), padz], 1).reshape(NW, NCH, CH)
        col2d = jnp.concatenate([col.reshape(NW, EPW), padz], 1).reshape(NW, NCH, CH)
    else:
        row2d = row.reshape(NW, NCH, CH)
        col2d = col.reshape(NW, NCH, CH)

    ones8 = jnp.ones((CH, 16), jnp.float32)
    zeros8 = jnp.zeros((NACC, 16), jnp.float32)
    zerosd = jnp.zeros((NACC, D), jnp.float32)
    b1r = b1.reshape(1, Dh)
    b2c = (b2 + jnp.asarray(cache_name, jnp.float32)).reshape(1, Do)

    mask_deg = _make_mask_deg_kernel(N, NCH, CH, NACC, STRIPE)
    propagate = _make_propagate_kernel(N, NCH, CH, D, NACC, STRIPE)

    mcol2d, degp = mask_deg(row2d, col2d, ones8, zeros8)

    BR = 1000
    grid = (N // BR,)
    degp_spec = pl.BlockSpec((NC, BR, 16), lambda i: (0, i, 0))
    row_spec = pl.BlockSpec((BR, Din), lambda i: (i, 0))
    part_spec = pl.BlockSpec((NC, BR, D), lambda i: (0, i, 0))
    w_spec = pl.BlockSpec((Din, Dh), lambda i: (0, 0))
    b_spec = pl.BlockSpec((1, Dh), lambda i: (0, 0))

    hs1 = pl.pallas_call(
        _tc_scale_matmul,
        grid=grid,
        in_specs=[row_spec, w_spec, degp_spec],
        out_specs=pl.BlockSpec((BR, Dh), lambda i: (i, 0)),
        out_shape=jax.ShapeDtypeStruct((N, Dh), jnp.float32),
    )(x, W1, degp)

    part1 = propagate(hs1, row2d, mcol2d, zerosd)

    hs2 = pl.pallas_call(
        _tc_combine_matmul,
        grid=grid,
        in_specs=[part_spec, row_spec, degp_spec, w_spec, b_spec],
        out_specs=pl.BlockSpec((BR, Do), lambda i: (i, 0)),
        out_shape=jax.ShapeDtypeStruct((N, Do), jnp.float32),
    )(part1, hs1, degp, W2, b1r)

    part2 = propagate(hs2, row2d, mcol2d, zerosd)

    out = pl.pallas_call(
        _tc_final,
        grid=grid,
        in_specs=[part_spec, row_spec, degp_spec, b_spec],
        out_specs=pl.BlockSpec((BR, Do), lambda i: (i, 0)),
        out_shape=jax.ShapeDtypeStruct((N, Do), jnp.float32),
    )(part2, hs2, degp, b2c)

    return out


# speedup vs baseline: 1.9970x; 1.0218x over previous
import functools

import jax
import jax.numpy as jnp
from jax import lax
from jax.experimental import pallas as pl
from jax.experimental.pallas import tpu as pltpu
from jax.experimental.pallas import tpu_sc as plsc

NC = 2
NS = 16
NW = NC * NS
LANES = 16


def _largest_chunk(epw):
    for c in range(128, 7, -8):
        if epw % c == 0:
            return c
    return 8


def _round_up(v, m):
    return -(-v // m) * m


def _make_mask_deg_kernel(N, NCH, CH, NACC, STRIPE):
    KSUB = CH // LANES
    mesh = plsc.VectorSubcoreMesh(core_axis_name="c", subcore_axis_name="s",
                                  num_cores=NC, num_subcores=NS)

    @functools.partial(
        pl.kernel,
        out_type=[
            jax.ShapeDtypeStruct((NW, NCH, CH), jnp.int32),
            jax.ShapeDtypeStruct((NC, NACC, 16), jnp.float32),
        ],
        mesh=mesh,
        compiler_params=pltpu.CompilerParams(use_tc_tiling_on_sc=False),
        scratch_types=[
            pltpu.VMEM((NCH, CH), jnp.int32),
            pltpu.VMEM((NCH, CH), jnp.int32),
            pltpu.VMEM((NCH, CH), jnp.int32),
            pltpu.VMEM((NCH, CH), jnp.int32),
            pltpu.VMEM((CH, 16), jnp.float32),
            pltpu.SemaphoreType.DMA,
            pltpu.VMEM_SHARED((NACC, 16), jnp.float32),
        ],
    )
    def mask_deg(row2d, col2d, ones8, zeros8, mcol_out, degp_out,
                 rowb, colb, mrowb, mcolb, onesb, semz, acc):
        cid = lax.axis_index("c")
        sid = lax.axis_index("s")
        wid = sid * NC + cid
        s0 = sid * STRIPE
        SPREAD = ((NACC - N) // LANES) * LANES
        iota = lax.iota(jnp.int32, LANES)

        z = pltpu.async_copy(zeros8.at[pl.ds(s0, STRIPE)],
                             acc.at[pl.ds(s0, STRIPE)], semz)
        r = pltpu.async_copy(row2d.at[wid], rowb, semz)
        c = pltpu.async_copy(col2d.at[wid], colb, semz)
        o = pltpu.async_copy(ones8, onesb, semz)
        z.wait()
        r.wait()
        c.wait()
        o.wait()

        def mask_body(j, _):
            for k in range(KSUB):
                sl = pl.ds(k * LANES, LANES)
                r = rowb[j, sl]
                c = colb[j, sl]
                is_self = r == c
                if SPREAD >= LANES:
                    base = lax.rem((j * KSUB + k) * LANES, SPREAD)
                    pad = (N + base) + iota
                else:
                    pad = jnp.full((LANES,), N, jnp.int32)
                mrowb[j, sl] = jnp.where(is_self, pad, r)
                mcolb[j, sl] = jnp.where(is_self, pad, c)
            return 0

        lax.fori_loop(0, NCH, mask_body, 0)
        pltpu.sync_copy(mcolb, mcol_out.at[wid])
        plsc.subcore_barrier()

        def scat_body(j, _):
            pltpu.sync_copy(onesb, acc.at[mrowb.at[j]], add=True)
            return 0

        lax.fori_loop(0, NCH, scat_body, 0)
        plsc.subcore_barrier()
        pltpu.sync_copy(acc.at[pl.ds(s0, STRIPE)],
                        degp_out.at[cid, pl.ds(s0, STRIPE)])

    return mask_deg


def _make_propagate_kernel(N, NCH, CH, D, NACC, STRIPE):
    mesh = plsc.VectorSubcoreMesh(core_axis_name="c", subcore_axis_name="s",
                                  num_cores=NC, num_subcores=NS)

    @functools.partial(
        pl.kernel,
        out_type=jax.ShapeDtypeStruct((NC, NACC, D), jnp.float32),
        mesh=mesh,
        compiler_params=pltpu.CompilerParams(use_tc_tiling_on_sc=False),
        scratch_types=[
            pltpu.VMEM((NCH, CH), jnp.int32),
            pltpu.VMEM((NCH, CH), jnp.int32),
            pltpu.VMEM((CH, D), jnp.float32),
            pltpu.VMEM((CH, D), jnp.float32),
            pltpu.VMEM((CH, D), jnp.float32),
            pltpu.SemaphoreType.DMA,
            pltpu.SemaphoreType.DMA,
            pltpu.SemaphoreType.DMA,
            pltpu.SemaphoreType.DMA,
            pltpu.VMEM_SHARED((NACC, D), jnp.float32),
        ],
    )
    def propagate(hs_hbm, row2d, mcol2d, zerosd, part_out,
                  ridx, cidx, rows0, rows1, rows2, semg0, semg1, semg2, sems,
                  acc):
        cid = lax.axis_index("c")
        sid = lax.axis_index("s")
        wid = sid * NC + cid
        s0 = sid * STRIPE
        rows = (rows0, rows1, rows2)
        semg = (semg0, semg1, semg2)

        z = pltpu.async_copy(zerosd.at[pl.ds(s0, STRIPE)],
                             acc.at[pl.ds(s0, STRIPE)], sems)
        r = pltpu.async_copy(row2d.at[wid], ridx, semg0)
        c = pltpu.async_copy(mcol2d.at[wid], cidx, semg1)
        z.wait()
        r.wait()
        c.wait()
        plsc.subcore_barrier()

        pltpu.async_copy(hs_hbm.at[ridx.at[0]], rows0, semg0)
        if NCH > 1:
            pltpu.async_copy(hs_hbm.at[ridx.at[1]], rows1, semg1)

        def step(j, _):
            for b in range(3):

                @pl.when(lax.rem(j, 3) == b)
                def _():
                    pltpu.make_async_copy(
                        hs_hbm.at[ridx.at[j]], rows[b], semg[b]).wait()

                    @pl.when(j > 0)
                    def _():
                        pltpu.make_async_copy(
                            rows[(b + 2) % 3], acc.at[cidx.at[j - 1]],
                            sems).wait()

                    @pl.when(j + 2 < NCH)
                    def _():
                        pltpu.async_copy(
                            hs_hbm.at[ridx.at[j + 2]], rows[(b + 2) % 3],
                            semg[(b + 2) % 3])
                    pltpu.async_copy(rows[b], acc.at[cidx.at[j]], sems,
                                     add=True)
            return 0

        lax.fori_loop(0, NCH, step, 0)
        pltpu.make_async_copy(rows[(NCH - 1) % 3], acc.at[cidx.at[NCH - 1]],
                              sems).wait()
        plsc.subcore_barrier()
        pltpu.sync_copy(acc.at[pl.ds(s0, STRIPE)],
                        part_out.at[cid, pl.ds(s0, STRIPE)])

    return propagate


def _dinv_block(degp):
    deg = degp[0, :, 0:1] + degp[1, :, 0:1] + 1.0
    return lax.rsqrt(deg)


def _tc_scale_matmul(x_ref, w_ref, degp_ref, out_ref):
    dinv = _dinv_block(degp_ref[...])
    h = jnp.dot(x_ref[...], w_ref[...], preferred_element_type=jnp.float32)
    out_ref[...] = h * dinv


def _tc_combine_matmul(part_ref, hs_ref, degp_ref, w_ref, b_ref, out_ref):
    dinv = _dinv_block(degp_ref[...])
    s = part_ref[0] + part_ref[1] + hs_ref[...]
    z = jnp.maximum(dinv * s + b_ref[...], 0.0)
    h = jnp.dot(z, w_ref[...], preferred_element_type=jnp.float32)
    out_ref[...] = h * dinv


def _tc_final(part_ref, hs_ref, degp_ref, b_ref, out_ref):
    dinv = _dinv_block(degp_ref[...])
    s = part_ref[0] + part_ref[1] + hs_ref[...]
    out_ref[...] = dinv * s + b_ref[...]


def kernel(x, edge_index, cache_name, W1, b1, W2, b2, Wd, bd):
    N, Din = x.shape
    Dh = W1.shape[1]
    Do = W2.shape[1]
    E = edge_index.shape[1]
    D = Dh

    EPW = E // NW
    CH = min(80, _round_up(EPW, 8))
    NCH = -(-EPW // CH)
    PADW = NCH * CH - EPW
    STRIPE = -(-(N + 1) // NS)
    NACC = STRIPE * NS

    row = edge_index[0].astype(jnp.int32)
    col = edge_index[1].astype(jnp.int32)
    if PADW:
        padz = jnp.zeros((NW, PADW), jnp.int32)
        row2d = jnp.concatenate([row.reshape(NW, EPW), padz], 1).reshape(NW, NCH, CH)
        col2d = jnp.concatenate([col.reshape(NW, EPW), padz], 1).reshape(NW, NCH, CH)
    else:
        row2d = row.reshape(NW, NCH, CH)
        col2d = col.reshape(NW, NCH, CH)

    ones8 = jnp.ones((CH, 16), jnp.float32)
    zeros8 = jnp.zeros((NACC, 16), jnp.float32)
    zerosd = jnp.zeros((NACC, D), jnp.float32)
    b1r = b1.reshape(1, Dh)
    b2c = (b2 + jnp.asarray(cache_name, jnp.float32)).reshape(1, Do)

    mask_deg = _make_mask_deg_kernel(N, NCH, CH, NACC, STRIPE)
    propagate = _make_propagate_kernel(N, NCH, CH, D, NACC, STRIPE)

    mcol2d, degp = mask_deg(row2d, col2d, ones8, zeros8)

    BR = 2000
    grid = (N // BR,)
    degp_spec = pl.BlockSpec((NC, BR, 16), lambda i: (0, i, 0))
    row_spec = pl.BlockSpec((BR, Din), lambda i: (i, 0))
    part_spec = pl.BlockSpec((NC, BR, D), lambda i: (0, i, 0))
    w_spec = pl.BlockSpec((Din, Dh), lambda i: (0, 0))
    b_spec = pl.BlockSpec((1, Dh), lambda i: (0, 0))

    hs1 = pl.pallas_call(
        _tc_scale_matmul,
        grid=grid,
        in_specs=[row_spec, w_spec, degp_spec],
        out_specs=pl.BlockSpec((BR, Dh), lambda i: (i, 0)),
        out_shape=jax.ShapeDtypeStruct((N, Dh), jnp.float32),
    )(x, W1, degp)

    part1 = propagate(hs1, row2d, mcol2d, zerosd)

    hs2 = pl.pallas_call(
        _tc_combine_matmul,
        grid=grid,
        in_specs=[part_spec, row_spec, degp_spec, w_spec, b_spec],
        out_specs=pl.BlockSpec((BR, Do), lambda i: (i, 0)),
        out_shape=jax.ShapeDtypeStruct((N, Do), jnp.float32),
    )(part1, hs1, degp, W2, b1r)

    part2 = propagate(hs2, row2d, mcol2d, zerosd)

    out = pl.pallas_call(
        _tc_final,
        grid=grid,
        in_specs=[part_spec, row_spec, degp_spec, b_spec],
        out_specs=pl.BlockSpec((BR, Do), lambda i: (i, 0)),
        out_shape=jax.ShapeDtypeStruct((N, Do), jnp.float32),
    )(part2, hs2, degp, b2c)

    return out
